# baseline (speedup 1.0000x reference)
import functools

import jax
import jax.numpy as jnp
from jax import lax
from jax.experimental import pallas as pl
from jax.experimental.pallas import tpu as pltpu
from jax.experimental.pallas import tpu_sc as plsc

L = 16
NC = 2
NS = 16
NW = NC * NS
SG = 4
QW = 4
NT = 1 << QW


def _make_sc_call(B, CTX, M, LOCAL):
    BW = B // NW
    NG = BW // SG
    NQ = CTX // QW
    MH = M // 2
    MBH = MH // L
    TSTRIDE = NQ * MH

    def body(tq_hbm, ppa_hbm, ppb_hbm, ip_hbm, out_hbm,
             tq_v, pp_v, ip_v, out_v):
        wid = lax.axis_index("c") * NS + lax.axis_index("s")
        base = wid * BW
        pltpu.sync_copy(tq_hbm.at[pl.ds(base, BW)], tq_v)
        pltpu.sync_copy(ip_hbm, ip_v)
        lane_iota = lax.iota(jnp.int32, L)

        for half in range(2):
            pltpu.sync_copy(ppa_hbm if half == 0 else ppb_hbm, pp_v)

            def group_body(g, carry):
                r0 = g * SG

                def chunk_body(qc, acc):
                    q0 = qc * L
                    tqrows = [tq_v[r0 + s, pl.ds(q0, L)] for s in range(SG)]
                    acc = list(acc)
                    for k in range(L):
                        for s in range(SG):
                            ib = jnp.broadcast_to(tqrows[s][k],
                                                  (L,)) + lane_iota
                            for j in range(MBH):
                                row = plsc.load_gather(pp_v, [ib + j * L])
                                acc[s * MBH + j] = acc[s * MBH + j] * row
                    return tuple(acc)

                init = tuple(jnp.full((L,), 1.0, jnp.float32)
                             for _ in range(SG * MBH))
                acc = lax.fori_loop(0, NQ // L, chunk_body, init)

                carry = list(carry)
                ip_rows = [[ip_v[l, pl.ds(half * MH + j * L, L)]
                            for j in range(MBH)] for l in range(LOCAL)]
                for s in range(SG):
                    lane = (g % (L // SG)) * SG + s
                    for l in range(LOCAL):
                        v = acc[s * MBH] * ip_rows[l][0]
                        for j in range(1, MBH):
                            v = v + acc[s * MBH + j] * ip_rows[l][j]
                        for sh in (8, 4, 2, 1):
                            perm = lane_iota ^ sh
                            v = v + v.at[perm].get(mode="promise_in_bounds")
                        carry[l] = jnp.where(lane_iota == lane, v, carry[l])

                @pl.when(g % (L // SG) == (L // SG) - 1)
                def _():
                    col = (g // (L // SG)) * L
                    for l in range(LOCAL):
                        if half == 0:
                            out_v[l, pl.ds(col, L)] = carry[l]
                        else:
                            out_v[l, pl.ds(col, L)] = (
                                out_v[l, pl.ds(col, L)] + carry[l])

                return tuple(carry)

            zero = jnp.zeros((L,), jnp.float32)
            lax.fori_loop(0, NG, group_body, (zero,) * LOCAL)

        for l in range(LOCAL):
            pltpu.sync_copy(out_v.at[l], out_hbm.at[l, pl.ds(base, BW)])

    mesh = plsc.VectorSubcoreMesh(core_axis_name="c", subcore_axis_name="s")
    return pl.kernel(
        body,
        mesh=mesh,
        compiler_params=pltpu.CompilerParams(needs_layout_passes=False),
        out_type=jax.ShapeDtypeStruct((LOCAL, B), jnp.float32),
        scratch_types=[
            pltpu.VMEM((BW, NQ), jnp.int32),
            pltpu.VMEM((NT * NQ * MH,), jnp.float32),
            pltpu.VMEM((LOCAL, M), jnp.float32),
            pltpu.VMEM((LOCAL, BW), jnp.float32),
        ],
    )


def kernel(context, context_param, inputs_param):
    LOCAL_N, M, CTX = context_param.shape
    B = context.shape[0]
    NQ = CTX // QW

    ctx_i = context.astype(jnp.int32).reshape(B, NQ, QW)
    weights = jnp.array([8, 4, 2, 1], jnp.int32)
    t4 = jnp.sum(ctx_i * weights, axis=-1)
    MH = M // 2
    tq = (t4 * (NQ * MH) + jnp.arange(NQ) * MH).astype(jnp.int32)

    cpT = jnp.transpose(context_param, (0, 2, 1))
    gq = cpT.reshape(LOCAL_N, NQ, QW, M)
    t_idx = jnp.arange(NT)
    pp = (gq[(t_idx >> 3) & 1, :, 0, :]
          * gq[(t_idx >> 2) & 1, :, 1, :]
          * gq[(t_idx >> 1) & 1, :, 2, :]
          * gq[t_idx & 1, :, 3, :])
    ppa = pp[:, :, : M // 2].reshape(-1)
    ppb = pp[:, :, M // 2:].reshape(-1)

    call = _make_sc_call(B, CTX, M, LOCAL_N)
    out_t = call(tq, ppa, ppb, inputs_param.astype(jnp.float32))
    return jnp.transpose(out_t)


# speedup vs baseline: 1.7675x; 1.7675x over previous
import functools

import jax
import jax.numpy as jnp
from jax import lax
from jax.experimental import pallas as pl
from jax.experimental.pallas import tpu as pltpu
from jax.experimental.pallas import tpu_sc as plsc

L = 16
NC = 2
NS = 16
NW = NC * NS
SG = 4
QW = 4
NT = 1 << QW


def _make_sc_call(B, CTX, M, LOCAL):
    BW = B // NW
    NG = BW // SG
    NQ = CTX // QW
    MH = M // 2
    MBH = MH // L
    TSTRIDE = NQ * MH

    def body(tq_hbm, ppa_hbm, ppb_hbm, ip_hbm, out_hbm,
             tq_v, pp_v, ip_v, out_v):
        wid = lax.axis_index("c") * NS + lax.axis_index("s")
        base = wid * BW
        pltpu.sync_copy(tq_hbm.at[pl.ds(base, BW)], tq_v)
        pltpu.sync_copy(ip_hbm, ip_v)
        lane_iota = lax.iota(jnp.int32, L)

        for half in range(2):
            pltpu.sync_copy(ppa_hbm if half == 0 else ppb_hbm, pp_v)
            ip_rows = [[ip_v[l, pl.ds(half * MH + j * L, L)]
                        for j in range(MBH)] for l in range(LOCAL)]

            def group_body(g, carry):
                r0 = g * SG

                def chunk_body(qc, acc):
                    q0 = qc * L
                    tqrows = [tq_v[r0 + s, pl.ds(q0, L)] for s in range(SG)]
                    acc = list(acc)
                    for k in range(L):
                        qvec = lane_iota + (q0 + k) * MH
                        for s in range(SG):
                            tb = jnp.broadcast_to(tqrows[s][k], (L,))
                            ib = tb * TSTRIDE + qvec
                            for j in range(MBH):
                                row = plsc.load_gather(pp_v, [ib + j * L])
                                acc[s * MBH + j] = acc[s * MBH + j] * row
                    return tuple(acc)

                init = tuple(jnp.full((L,), 1.0, jnp.float32)
                             for _ in range(SG * MBH))
                acc = lax.fori_loop(0, NQ // L, chunk_body, init)

                carry = list(carry)
                for s in range(SG):
                    lane = (g % (L // SG)) * SG + s
                    for l in range(LOCAL):
                        v = acc[s * MBH] * ip_rows[l][0]
                        for j in range(1, MBH):
                            v = v + acc[s * MBH + j] * ip_rows[l][j]
                        for sh in (8, 4, 2, 1):
                            perm = lane_iota ^ sh
                            v = v + v.at[perm].get(mode="promise_in_bounds")
                        carry[l] = jnp.where(lane_iota == lane, v, carry[l])

                @pl.when(g % (L // SG) == (L // SG) - 1)
                def _():
                    col = (g // (L // SG)) * L
                    for l in range(LOCAL):
                        if half == 0:
                            out_v[l, pl.ds(col, L)] = carry[l]
                        else:
                            out_v[l, pl.ds(col, L)] = (
                                out_v[l, pl.ds(col, L)] + carry[l])

                return tuple(carry)

            zero = jnp.zeros((L,), jnp.float32)
            lax.fori_loop(0, NG, group_body, (zero,) * LOCAL)

        for l in range(LOCAL):
            pltpu.sync_copy(out_v.at[l], out_hbm.at[l, pl.ds(base, BW)])

    mesh = plsc.VectorSubcoreMesh(core_axis_name="c", subcore_axis_name="s")
    return pl.kernel(
        body,
        mesh=mesh,
        compiler_params=pltpu.CompilerParams(needs_layout_passes=False),
        out_type=jax.ShapeDtypeStruct((LOCAL, B), jnp.float32),
        scratch_types=[
            pltpu.VMEM((BW, NQ), jnp.int32),
            pltpu.VMEM((NT * NQ * MH,), jnp.float32),
            pltpu.VMEM((LOCAL, M), jnp.float32),
            pltpu.VMEM((LOCAL, BW), jnp.float32),
        ],
    )


def kernel(context, context_param, inputs_param):
    LOCAL_N, M, CTX = context_param.shape
    B = context.shape[0]
    NQ = CTX // QW

    ctx_i = context.astype(jnp.int32).reshape(B, NQ, QW)
    weights = jnp.array([8, 4, 2, 1], jnp.int32)
    tq = jnp.sum(ctx_i * weights, axis=-1).astype(jnp.int32)

    cpT = jnp.transpose(context_param, (0, 2, 1))
    gq = cpT.reshape(LOCAL_N, NQ, QW, M)
    t_idx = jnp.arange(NT)
    pp = (gq[(t_idx >> 3) & 1, :, 0, :]
          * gq[(t_idx >> 2) & 1, :, 1, :]
          * gq[(t_idx >> 1) & 1, :, 2, :]
          * gq[t_idx & 1, :, 3, :])
    ppa = pp[:, :, : M // 2].reshape(-1)
    ppb = pp[:, :, M // 2:].reshape(-1)

    call = _make_sc_call(B, CTX, M, LOCAL_N)
    out_t = call(tq, ppa, ppb, inputs_param.astype(jnp.float32))
    return jnp.transpose(out_t)
